# baseline (speedup 1.0000x reference)
import functools

import jax
import jax.numpy as jnp
from jax import lax
from jax.experimental import pallas as pl
from jax.experimental.pallas import tpu as pltpu
from jax.experimental.pallas import tpu_sc as plsc

EMBED_DIM = 64
NUM_WORKERS = 32
NBUF = 2
ROWS_PER_CHUNK = 8


def _sc_gather(indices, table):
    batch, hist = indices.shape
    rows_pw = batch // NUM_WORKERS
    n_chunks = rows_pw // ROWS_PER_CHUNK
    assert n_chunks % NBUF == 0 and n_chunks // NBUF >= 2
    mesh = plsc.VectorSubcoreMesh(core_axis_name="c", subcore_axis_name="s")

    chunk = ROWS_PER_CHUNK * hist

    @functools.partial(
        pl.kernel,
        mesh=mesh,
        out_type=jax.ShapeDtypeStruct((batch * hist, 128), jnp.float32),
        scratch_types=[
            pltpu.VMEM((rows_pw, hist), jnp.int32),
            *[pltpu.VMEM((ROWS_PER_CHUNK * hist, EMBED_DIM), jnp.float32)
              for _ in range(NBUF)],
            *[pltpu.SemaphoreType.DMA for _ in range(2 * NBUF)],
        ],
        compiler_params=pltpu.CompilerParams(use_tc_tiling_on_sc=False),
    )
    def grab(idx_hbm, table_hbm, out_hbm, idx_v, *bufs_and_sems):
        rows = bufs_and_sems[:NBUF]
        gsem = bufs_and_sems[NBUF:2 * NBUF]
        osem = bufs_and_sems[2 * NBUF:]
        wid = lax.axis_index("s") * 2 + lax.axis_index("c")
        base = wid * rows_pw
        pltpu.sync_copy(idx_hbm.at[pl.ds(base, rows_pw), :], idx_v)

        def gather(j, b):
            for r in range(ROWS_PER_CHUNK):
                pltpu.async_copy(
                    table_hbm.at[idx_v.at[j * ROWS_PER_CHUNK + r]],
                    rows[b].at[pl.ds(r * hist, hist), :], gsem[b])

        def put(j, b):
            pltpu.async_copy(
                rows[b],
                out_hbm.at[pl.ds((base + j * ROWS_PER_CHUNK) * hist, chunk),
                           pl.ds(0, EMBED_DIM)],
                osem[b])

        def wait_gather(b):
            pltpu.make_async_copy(
                out_hbm.at[pl.ds(0, chunk), pl.ds(0, EMBED_DIM)], rows[b],
                gsem[b]).wait()

        def wait_put(b):
            pltpu.make_async_copy(
                rows[b],
                out_hbm.at[pl.ds(base * hist, chunk), pl.ds(0, EMBED_DIM)],
                osem[b]).wait()

        for b in range(NBUF):
            gather(b, b)

        def body(i, carry):
            j0 = NBUF * i
            for b in range(NBUF):
                wait_gather(b)
                put(j0 + b, b)

                @pl.when(i + 1 < n_chunks // NBUF)
                def _():
                    wait_put(b)
                    gather(j0 + b + NBUF, b)

            return carry

        lax.fori_loop(0, n_chunks // NBUF, body, 0)
        for b in range(NBUF):
            wait_put(b)

    return grab(indices, table)


def kernel(indices, table):
    batch, hist = indices.shape
    out_fat = _sc_gather(indices.astype(jnp.int32), table)
    return out_fat[:, :EMBED_DIM].reshape(batch, hist, EMBED_DIM)


# speedup vs baseline: 1.4386x; 1.4386x over previous
import functools

import jax
import jax.numpy as jnp
from jax import lax
from jax.experimental import pallas as pl
from jax.experimental.pallas import tpu as pltpu
from jax.experimental.pallas import tpu_sc as plsc

EMBED_DIM = 64
NUM_WORKERS = 32
NBUF = 2
ROWS_PER_CHUNK = 8


def _sc_gather(indices, table):
    batch, hist = indices.shape
    rows_pw = batch // NUM_WORKERS
    n_chunks = rows_pw // ROWS_PER_CHUNK
    assert n_chunks % NBUF == 0 and n_chunks // NBUF >= 2
    mesh = plsc.VectorSubcoreMesh(core_axis_name="c", subcore_axis_name="s")

    @functools.partial(
        pl.kernel,
        mesh=mesh,
        out_type=jax.ShapeDtypeStruct((batch, 56, 128), jnp.float32),
        scratch_types=[
            pltpu.VMEM((rows_pw, hist), jnp.int32),
            *[pltpu.VMEM((ROWS_PER_CHUNK, hist, EMBED_DIM), jnp.float32)
              for _ in range(NBUF)],
            *[pltpu.SemaphoreType.DMA for _ in range(2 * NBUF)],
        ],
        compiler_params=pltpu.CompilerParams(use_tc_tiling_on_sc=False),
    )
    def grab(idx_hbm, table_hbm, out_hbm, idx_v, *bufs_and_sems):
        rows = bufs_and_sems[:NBUF]
        gsem = bufs_and_sems[NBUF:2 * NBUF]
        osem = bufs_and_sems[2 * NBUF:]
        wid = lax.axis_index("s") * 2 + lax.axis_index("c")
        base = wid * rows_pw
        pltpu.sync_copy(idx_hbm.at[pl.ds(base, rows_pw), :], idx_v)

        def gather(j, b):
            for r in range(ROWS_PER_CHUNK):
                pltpu.async_copy(
                    table_hbm.at[idx_v.at[j * ROWS_PER_CHUNK + r]],
                    rows[b].at[r], gsem[b])

        def put(j, b):
            pltpu.async_copy(
                rows[b],
                out_hbm.at[pl.ds(base + j * ROWS_PER_CHUNK, ROWS_PER_CHUNK),
                           pl.ds(0, hist), pl.ds(0, EMBED_DIM)],
                osem[b])

        def wait_gather(b):
            pltpu.make_async_copy(
                out_hbm.at[pl.ds(0, ROWS_PER_CHUNK), pl.ds(0, hist),
                           pl.ds(0, EMBED_DIM)],
                rows[b], gsem[b]).wait()

        def wait_put(b):
            pltpu.make_async_copy(
                rows[b],
                out_hbm.at[pl.ds(base, ROWS_PER_CHUNK), pl.ds(0, hist),
                           pl.ds(0, EMBED_DIM)],
                osem[b]).wait()

        for b in range(NBUF):
            gather(b, b)

        def body(i, carry):
            j0 = NBUF * i
            for b in range(NBUF):
                wait_gather(b)
                put(j0 + b, b)

                @pl.when(i + 1 < n_chunks // NBUF)
                def _():
                    wait_put(b)
                    gather(j0 + b + NBUF, b)

            return carry

        lax.fori_loop(0, n_chunks // NBUF, body, 0)
        for b in range(NBUF):
            wait_put(b)

    return grab(indices, table)


def kernel(indices, table):
    hist = indices.shape[1]
    return _sc_gather(indices.astype(jnp.int32), table)[:, :hist, :EMBED_DIM]


# speedup vs baseline: 1.4403x; 1.0012x over previous
import functools

import jax
import jax.numpy as jnp
from jax import lax
from jax.experimental import pallas as pl
from jax.experimental.pallas import tpu as pltpu
from jax.experimental.pallas import tpu_sc as plsc

EMBED_DIM = 64
NUM_WORKERS = 32
NBUF = 2
ROWS_PER_CHUNK = 16


def _sc_gather(indices, table):
    batch, hist = indices.shape
    rows_pw = batch // NUM_WORKERS
    n_chunks = rows_pw // ROWS_PER_CHUNK
    assert n_chunks % NBUF == 0 and n_chunks // NBUF >= 2
    mesh = plsc.VectorSubcoreMesh(core_axis_name="c", subcore_axis_name="s")

    @functools.partial(
        pl.kernel,
        mesh=mesh,
        out_type=jax.ShapeDtypeStruct((batch, 56, 128), jnp.float32),
        scratch_types=[
            pltpu.VMEM((rows_pw, hist), jnp.int32),
            *[pltpu.VMEM((ROWS_PER_CHUNK, hist, EMBED_DIM), jnp.float32)
              for _ in range(NBUF)],
            *[pltpu.SemaphoreType.DMA for _ in range(2 * NBUF)],
        ],
        compiler_params=pltpu.CompilerParams(use_tc_tiling_on_sc=False),
    )
    def grab(idx_hbm, table_hbm, out_hbm, idx_v, *bufs_and_sems):
        rows = bufs_and_sems[:NBUF]
        gsem = bufs_and_sems[NBUF:2 * NBUF]
        osem = bufs_and_sems[2 * NBUF:]
        wid = lax.axis_index("s") * 2 + lax.axis_index("c")
        base = wid * rows_pw
        pltpu.sync_copy(idx_hbm.at[pl.ds(base, rows_pw), :], idx_v)

        def gather(j, b):
            for r in range(ROWS_PER_CHUNK):
                pltpu.async_copy(
                    table_hbm.at[idx_v.at[j * ROWS_PER_CHUNK + r]],
                    rows[b].at[r], gsem[b])

        def put(j, b):
            pltpu.async_copy(
                rows[b],
                out_hbm.at[pl.ds(base + j * ROWS_PER_CHUNK, ROWS_PER_CHUNK),
                           pl.ds(0, hist), pl.ds(0, EMBED_DIM)],
                osem[b])

        def wait_gather(b):
            pltpu.make_async_copy(
                out_hbm.at[pl.ds(0, ROWS_PER_CHUNK), pl.ds(0, hist),
                           pl.ds(0, EMBED_DIM)],
                rows[b], gsem[b]).wait()

        def wait_put(b):
            pltpu.make_async_copy(
                rows[b],
                out_hbm.at[pl.ds(base, ROWS_PER_CHUNK), pl.ds(0, hist),
                           pl.ds(0, EMBED_DIM)],
                osem[b]).wait()

        for b in range(NBUF):
            gather(b, b)

        def body(i, carry):
            j0 = NBUF * i
            for b in range(NBUF):
                wait_gather(b)
                put(j0 + b, b)

                @pl.when(i + 1 < n_chunks // NBUF)
                def _():
                    wait_put(b)
                    gather(j0 + b + NBUF, b)

            return carry

        lax.fori_loop(0, n_chunks // NBUF, body, 0)
        for b in range(NBUF):
            wait_put(b)

    return grab(indices, table)


def kernel(indices, table):
    hist = indices.shape[1]
    return _sc_gather(indices.astype(jnp.int32), table)[:, :hist, :EMBED_DIM]
